# baseline (speedup 1.0000x reference)
import jax
import jax.numpy as jnp
from jax.experimental import pallas as pl
from jax.experimental.pallas import tpu as pltpu

_ROWS_PER_TILE = 2048


def _mod_kernel(idx_ref, emb_hbm, W_ref, b_ref, out_ref, row_vmem, dma_sem):
    i = pl.program_id(0)

    @pl.when(i == 0)
    def _fetch():
        cp = pltpu.make_async_copy(
            emb_hbm.at[pl.ds(idx_ref[0], 1), :], row_vmem, dma_sem)
        cp.start()
        cp.wait()

    code = row_vmem[0, :]
    mods = jnp.sum(W_ref[...] * code[None, :], axis=1) + b_ref[0, :]
    out_ref[...] = jnp.broadcast_to(mods[None, :], out_ref.shape)


def kernel(coords, patient_idx, emb_table, W, b):
    n = coords.shape[0]
    num_out, code_dim = W.shape
    idx = jnp.asarray(patient_idx, jnp.int32).reshape((1,))
    grid = (n // _ROWS_PER_TILE,)
    out = pl.pallas_call(
        _mod_kernel,
        grid_spec=pltpu.PrefetchScalarGridSpec(
            num_scalar_prefetch=1,
            grid=grid,
            in_specs=[
                pl.BlockSpec(memory_space=pl.ANY),
                pl.BlockSpec((num_out, code_dim), lambda i, idx_ref: (0, 0)),
                pl.BlockSpec((1, num_out), lambda i, idx_ref: (0, 0)),
            ],
            out_specs=pl.BlockSpec((_ROWS_PER_TILE, num_out), lambda i, idx_ref: (i, 0)),
            scratch_shapes=[
                pltpu.VMEM((1, code_dim), jnp.float32),
                pltpu.SemaphoreType.DMA,
            ],
        ),
        out_shape=jax.ShapeDtypeStruct((n, num_out), jnp.float32),
    )(idx, emb_table, W, b.reshape(1, num_out))
    return out


# speedup vs baseline: 34.0219x; 34.0219x over previous
import jax
import jax.numpy as jnp
from jax.experimental import pallas as pl
from jax.experimental.pallas import tpu as pltpu

_ROWS_PER_TILE = 2048


def _mod_kernel(row_ref, W_ref, b_ref, out_ref):
    code = row_ref[0, :]
    mods = jnp.sum(W_ref[...] * code[None, :], axis=1) + b_ref[0, :]
    out_ref[...] = jnp.broadcast_to(mods[None, :], out_ref.shape)


def kernel(coords, patient_idx, emb_table, W, b):
    n = coords.shape[0]
    num_out, code_dim = W.shape
    idx = jnp.asarray(patient_idx, jnp.int32)
    row = jax.lax.dynamic_slice_in_dim(emb_table, idx, 1, axis=0)
    grid = (n // _ROWS_PER_TILE,)
    out = pl.pallas_call(
        _mod_kernel,
        grid=grid,
        in_specs=[
            pl.BlockSpec((1, code_dim), lambda i: (0, 0)),
            pl.BlockSpec((num_out, code_dim), lambda i: (0, 0)),
            pl.BlockSpec((1, num_out), lambda i: (0, 0)),
        ],
        out_specs=pl.BlockSpec((_ROWS_PER_TILE, num_out), lambda i: (i, 0)),
        out_shape=jax.ShapeDtypeStruct((n, num_out), jnp.float32),
    )(row, W, b.reshape(1, num_out))
    return out
